# baseline (speedup 1.0000x reference)
import jax
import jax.numpy as jnp
from jax import lax
from jax.experimental import pallas as pl
from jax.experimental.pallas import tpu as pltpu
from jax.experimental.pallas import tpu_sc as plsc

N_NODES = 10000
D = 128
N_CORES = 2
N_SUBCORES = 16
N_WORKERS = N_CORES * N_SUBCORES
EDGES_PER_TILE = 10000
CHUNK = 128
N_CHUNKS = EDGES_PER_TILE // CHUNK
REM = EDGES_PER_TILE - N_CHUNKS * CHUNK
ROWS_PER_TILE = 632
N_PAD = N_SUBCORES * ROWS_PER_TILE


def _mo(v):
    return pl.multiple_of(v, 8)


def _sc_body(x_hbm, src_hbm, dst_hbm, out_hbm,
             sidx, dch0, dch1, dchr, rows0, rows1, rowsr, acc,
             semr0, semr1, semd0, semd1, sems0, sems1, semrr, semdr):
    c = lax.axis_index("c")
    s = lax.axis_index("s")
    wid = c * N_SUBCORES + s
    ebase = _mo(wid * EDGES_PER_TILE)

    pltpu.sync_copy(src_hbm.at[pl.ds(ebase, EDGES_PER_TILE)], sidx)

    zero = jnp.zeros((16,), jnp.float32)

    def _zrow(i, carry):
        for j in range(D // 16):
            rows0[i, pl.ds(j * 16, 16)] = zero
        return carry

    lax.fori_loop(0, CHUNK, _zrow, 0)
    abase = _mo(s * ROWS_PER_TILE)
    for k in range(ROWS_PER_TILE // CHUNK):
        pltpu.sync_copy(rows0, acc.at[pl.ds(_mo(abase + k * CHUNK), CHUNK), :])
    zrem = ROWS_PER_TILE % CHUNK
    pltpu.sync_copy(rows0.at[pl.ds(0, zrem), :],
                    acc.at[pl.ds(_mo(abase + ROWS_PER_TILE - zrem), zrem), :])
    plsc.subcore_barrier()

    def _start(j, rows, dch, semr, semd):
        off = _mo(j * CHUNK)
        pltpu.async_copy(x_hbm.at[sidx.at[pl.ds(off, CHUNK)]], rows, semr)
        pltpu.async_copy(dst_hbm.at[pl.ds(ebase + off, CHUNK)], dch, semd)

    def _wait_in(rows, dch, semr, semd):
        pltpu.make_async_copy(x_hbm.at[sidx.at[pl.ds(0, CHUNK)]], rows, semr).wait()
        pltpu.make_async_copy(dst_hbm.at[pl.ds(0, CHUNK)], dch, semd).wait()

    def _wait_sc(rows, dch, sems):
        pltpu.make_async_copy(rows, acc.at[dch], sems).wait()

    _start(0, rows0, dch0, semr0, semd0)
    _start(1, rows1, dch1, semr1, semd1)

    def _pair(k, carry):
        j0 = k * 2
        _wait_in(rows0, dch0, semr0, semd0)
        pltpu.async_copy(rows0, acc.at[dch0], sems0, add=True)
        _wait_in(rows1, dch1, semr1, semd1)
        pltpu.async_copy(rows1, acc.at[dch1], sems1, add=True)
        _wait_sc(rows0, dch0, sems0)
        _start(j0 + 2, rows0, dch0, semr0, semd0)
        _wait_sc(rows1, dch1, sems1)
        _start(j0 + 3, rows1, dch1, semr1, semd1)
        return carry

    lax.fori_loop(0, N_CHUNKS // 2 - 1, _pair, 0)
    roff = _mo(N_CHUNKS * CHUNK)
    pltpu.async_copy(x_hbm.at[sidx.at[pl.ds(roff, REM)]], rowsr, semrr)
    pltpu.async_copy(dst_hbm.at[pl.ds(ebase + roff, REM)], dchr, semdr)
    _wait_in(rows0, dch0, semr0, semd0)
    pltpu.async_copy(rows0, acc.at[dch0], sems0, add=True)
    _wait_in(rows1, dch1, semr1, semd1)
    pltpu.async_copy(rows1, acc.at[dch1], sems1, add=True)
    pltpu.make_async_copy(x_hbm.at[sidx.at[pl.ds(0, REM)]], rowsr, semrr).wait()
    pltpu.make_async_copy(dst_hbm.at[pl.ds(0, REM)], dchr, semdr).wait()
    pltpu.async_copy(rowsr, acc.at[dchr], semrr, add=True)
    _wait_sc(rows0, dch0, sems0)
    _wait_sc(rows1, dch1, sems1)
    pltpu.make_async_copy(rowsr, acc.at[dchr], semrr).wait()
    plsc.subcore_barrier()

    pltpu.sync_copy(acc.at[pl.ds(abase, ROWS_PER_TILE), :],
                    out_hbm.at[c, pl.ds(abase, ROWS_PER_TILE), :])


_sc_segsum = pl.kernel(
    _sc_body,
    out_type=jax.ShapeDtypeStruct((N_CORES, N_PAD, D), jnp.float32),
    mesh=plsc.VectorSubcoreMesh(core_axis_name="c", subcore_axis_name="s"),
    scratch_types=[
        pltpu.VMEM((EDGES_PER_TILE,), jnp.int32),
        pltpu.VMEM((CHUNK,), jnp.int32),
        pltpu.VMEM((CHUNK,), jnp.int32),
        pltpu.VMEM((REM,), jnp.int32),
        pltpu.VMEM((CHUNK, D), jnp.float32),
        pltpu.VMEM((CHUNK, D), jnp.float32),
        pltpu.VMEM((REM, D), jnp.float32),
        pltpu.VMEM_SHARED((N_PAD, D), jnp.float32),
        pltpu.SemaphoreType.DMA,
        pltpu.SemaphoreType.DMA,
        pltpu.SemaphoreType.DMA,
        pltpu.SemaphoreType.DMA,
        pltpu.SemaphoreType.DMA,
        pltpu.SemaphoreType.DMA,
        pltpu.SemaphoreType.DMA,
        pltpu.SemaphoreType.DMA,
    ],
)


def _mm_body(p_ref, w_ref, b_ref, o_ref):
    h = p_ref[0] + p_ref[1]
    o_ref[...] = lax.dot_general(
        h, w_ref[...], (((1,), (1,)), ((), ())),
        preferred_element_type=jnp.float32) + b_ref[...]


ROW_BLK = 1000

_mm = pl.pallas_call(
    _mm_body,
    grid=(N_NODES // ROW_BLK,),
    in_specs=[
        pl.BlockSpec((N_CORES, ROW_BLK, D), lambda i: (0, i, 0)),
        pl.BlockSpec((D, D), lambda i: (0, 0)),
        pl.BlockSpec((1, D), lambda i: (0, 0)),
    ],
    out_specs=pl.BlockSpec((ROW_BLK, D), lambda i: (i, 0)),
    out_shape=jax.ShapeDtypeStruct((N_NODES, D), jnp.float32),
)


@jax.jit
def kernel(x, edge_index, W, b):
    src = edge_index[0]
    dst = edge_index[1]
    parts = _sc_segsum(x, src, dst)
    return _mm(parts, W, b.reshape(1, D))


# speedup vs baseline: 1.2549x; 1.2549x over previous
import jax
import jax.numpy as jnp
from jax import lax
from jax.experimental import pallas as pl
from jax.experimental.pallas import tpu as pltpu
from jax.experimental.pallas import tpu_sc as plsc

N_NODES = 10000
D = 128
N_CORES = 2
N_SUBCORES = 16
N_WORKERS = N_CORES * N_SUBCORES
EDGES_PER_TILE = 10000
CHUNK = 128
N_CHUNKS = EDGES_PER_TILE // CHUNK
REM = EDGES_PER_TILE - N_CHUNKS * CHUNK
ROWS_PER_TILE = 632
N_PAD = N_SUBCORES * ROWS_PER_TILE


def _mo(v):
    return pl.multiple_of(v, 8)


def _sc_body(x_hbm, src_hbm, dst_hbm, out_hbm,
             sidx, dch0, dch1, dchr, rows0, rows1, rowsr, acc,
             semr0, semr1, semd0, semd1, sems0, sems1, semrr, semdr):
    c = lax.axis_index("c")
    s = lax.axis_index("s")
    wid = c * N_SUBCORES + s
    ebase = _mo(wid * EDGES_PER_TILE)

    pltpu.sync_copy(src_hbm.at[pl.ds(ebase, EDGES_PER_TILE)], sidx)

    zero = jnp.zeros((16,), jnp.float32)

    def _zrow(i, carry):
        for j in range(D // 16):
            rows0[i, pl.ds(j * 16, 16)] = zero
        return carry

    lax.fori_loop(0, CHUNK, _zrow, 0)
    abase = _mo(s * ROWS_PER_TILE)
    for k in range(ROWS_PER_TILE // CHUNK):
        pltpu.sync_copy(rows0, acc.at[pl.ds(_mo(abase + k * CHUNK), CHUNK), :])
    zrem = ROWS_PER_TILE % CHUNK
    pltpu.sync_copy(rows0.at[pl.ds(0, zrem), :],
                    acc.at[pl.ds(_mo(abase + ROWS_PER_TILE - zrem), zrem), :])
    plsc.subcore_barrier()

    def _start(j, rows, dch, semr, semd):
        off = _mo(j * CHUNK)
        pltpu.async_copy(x_hbm.at[sidx.at[pl.ds(off, CHUNK)]], rows, semr)
        pltpu.async_copy(dst_hbm.at[pl.ds(ebase + off, CHUNK)], dch, semd)

    def _wait_in(rows, dch, semr, semd):
        pltpu.make_async_copy(x_hbm.at[sidx.at[pl.ds(0, CHUNK)]], rows, semr).wait()
        pltpu.make_async_copy(dst_hbm.at[pl.ds(0, CHUNK)], dch, semd).wait()

    def _wait_sc(rows, dch, sems):
        pltpu.make_async_copy(rows, acc.at[dch], sems).wait()

    _start(0, rows0, dch0, semr0, semd0)
    _start(1, rows1, dch1, semr1, semd1)

    def _pair(k, carry):
        j0 = k * 2
        _wait_in(rows0, dch0, semr0, semd0)
        pltpu.sync_copy(rows0, acc.at[dch0], add=True)
        _start(j0 + 2, rows0, dch0, semr0, semd0)
        _wait_in(rows1, dch1, semr1, semd1)
        pltpu.sync_copy(rows1, acc.at[dch1], add=True)
        _start(j0 + 3, rows1, dch1, semr1, semd1)
        return carry

    lax.fori_loop(0, N_CHUNKS // 2 - 1, _pair, 0)
    roff = _mo(N_CHUNKS * CHUNK)
    pltpu.async_copy(x_hbm.at[sidx.at[pl.ds(roff, REM)]], rowsr, semrr)
    pltpu.async_copy(dst_hbm.at[pl.ds(ebase + roff, REM)], dchr, semdr)
    _wait_in(rows0, dch0, semr0, semd0)
    pltpu.sync_copy(rows0, acc.at[dch0], add=True)
    _wait_in(rows1, dch1, semr1, semd1)
    pltpu.sync_copy(rows1, acc.at[dch1], add=True)
    pltpu.make_async_copy(x_hbm.at[sidx.at[pl.ds(0, REM)]], rowsr, semrr).wait()
    pltpu.make_async_copy(dst_hbm.at[pl.ds(0, REM)], dchr, semdr).wait()
    pltpu.sync_copy(rowsr, acc.at[dchr], add=True)
    plsc.subcore_barrier()

    pltpu.sync_copy(acc.at[pl.ds(abase, ROWS_PER_TILE), :],
                    out_hbm.at[c, pl.ds(abase, ROWS_PER_TILE), :])


_sc_segsum = pl.kernel(
    _sc_body,
    out_type=jax.ShapeDtypeStruct((N_CORES, N_PAD, D), jnp.float32),
    mesh=plsc.VectorSubcoreMesh(core_axis_name="c", subcore_axis_name="s"),
    scratch_types=[
        pltpu.VMEM((EDGES_PER_TILE,), jnp.int32),
        pltpu.VMEM((CHUNK,), jnp.int32),
        pltpu.VMEM((CHUNK,), jnp.int32),
        pltpu.VMEM((REM,), jnp.int32),
        pltpu.VMEM((CHUNK, D), jnp.float32),
        pltpu.VMEM((CHUNK, D), jnp.float32),
        pltpu.VMEM((REM, D), jnp.float32),
        pltpu.VMEM_SHARED((N_PAD, D), jnp.float32),
        pltpu.SemaphoreType.DMA,
        pltpu.SemaphoreType.DMA,
        pltpu.SemaphoreType.DMA,
        pltpu.SemaphoreType.DMA,
        pltpu.SemaphoreType.DMA,
        pltpu.SemaphoreType.DMA,
        pltpu.SemaphoreType.DMA,
        pltpu.SemaphoreType.DMA,
    ],
)


def _mm_body(p_ref, w_ref, b_ref, o_ref):
    h = p_ref[0] + p_ref[1]
    o_ref[...] = lax.dot_general(
        h, w_ref[...], (((1,), (1,)), ((), ())),
        preferred_element_type=jnp.float32) + b_ref[...]


ROW_BLK = 1000

_mm = pl.pallas_call(
    _mm_body,
    grid=(N_NODES // ROW_BLK,),
    in_specs=[
        pl.BlockSpec((N_CORES, ROW_BLK, D), lambda i: (0, i, 0)),
        pl.BlockSpec((D, D), lambda i: (0, 0)),
        pl.BlockSpec((1, D), lambda i: (0, 0)),
    ],
    out_specs=pl.BlockSpec((ROW_BLK, D), lambda i: (i, 0)),
    out_shape=jax.ShapeDtypeStruct((N_NODES, D), jnp.float32),
)


@jax.jit
def kernel(x, edge_index, W, b):
    src = edge_index[0]
    dst = edge_index[1]
    parts = _sc_segsum(x, src, dst)
    return _mm(parts, W, b.reshape(1, D))


# speedup vs baseline: 1.3945x; 1.1113x over previous
import jax
import jax.numpy as jnp
from jax import lax
from jax.experimental import pallas as pl
from jax.experimental.pallas import tpu as pltpu
from jax.experimental.pallas import tpu_sc as plsc

N_NODES = 10000
N_EDGES = 320000
D = 128
N_CORES = 2
N_SUBCORES = 16
N_WORKERS = N_CORES * N_SUBCORES
EDGES_PER_TILE = 10000
CHUNK = 128
N_CHUNKS = EDGES_PER_TILE // CHUNK
REM = EDGES_PER_TILE - N_CHUNKS * CHUNK
ROWS_PER_TILE = 632
N_PAD = N_SUBCORES * ROWS_PER_TILE


def _mo(v):
    return pl.multiple_of(v, 8)


def _sc_body(x_hbm, edge_hbm, out_hbm,
             sidx, dch0, dch1, dchr, rows0, rows1, rowsr, acc,
             semr0, semr1, semd0, semd1, sems0, sems1, semrr, semdr):
    c = lax.axis_index("c")
    s = lax.axis_index("s")
    wid = c * N_SUBCORES + s
    ebase = _mo(wid * EDGES_PER_TILE)
    dbase = _mo(N_EDGES + wid * EDGES_PER_TILE)

    pltpu.async_copy(edge_hbm.at[pl.ds(ebase, EDGES_PER_TILE)], sidx, semr0)

    zero = jnp.zeros((16,), jnp.float32)

    def _zrow(i, carry):
        for j in range(D // 16):
            rows0[i, pl.ds(j * 16, 16)] = zero
        return carry

    lax.fori_loop(0, CHUNK, _zrow, 0)
    abase = _mo(s * ROWS_PER_TILE)
    for k in range(ROWS_PER_TILE // CHUNK):
        pltpu.async_copy(rows0, acc.at[pl.ds(_mo(abase + k * CHUNK), CHUNK), :], sems0)
    zrem = ROWS_PER_TILE % CHUNK
    pltpu.async_copy(rows0.at[pl.ds(0, zrem), :],
                     acc.at[pl.ds(_mo(abase + ROWS_PER_TILE - zrem), zrem), :], sems1)
    for k in range(ROWS_PER_TILE // CHUNK):
        pltpu.make_async_copy(rows0, acc.at[pl.ds(abase, CHUNK), :], sems0).wait()
    pltpu.make_async_copy(rows0.at[pl.ds(0, zrem), :],
                          acc.at[pl.ds(abase, zrem), :], sems1).wait()
    pltpu.make_async_copy(edge_hbm.at[pl.ds(0, EDGES_PER_TILE)], sidx, semr0).wait()
    plsc.subcore_barrier()

    def _start(j, rows, dch, semr, semd):
        off = _mo(j * CHUNK)
        pltpu.async_copy(x_hbm.at[sidx.at[pl.ds(off, CHUNK)]], rows, semr)
        pltpu.async_copy(edge_hbm.at[pl.ds(dbase + off, CHUNK)], dch, semd)

    def _wait_in(rows, dch, semr, semd):
        pltpu.make_async_copy(x_hbm.at[sidx.at[pl.ds(0, CHUNK)]], rows, semr).wait()
        pltpu.make_async_copy(edge_hbm.at[pl.ds(0, CHUNK)], dch, semd).wait()

    def _wait_sc(rows, dch, sems):
        pltpu.make_async_copy(rows, acc.at[dch], sems).wait()

    _start(0, rows0, dch0, semr0, semd0)
    _start(1, rows1, dch1, semr1, semd1)

    def _pair(k, carry):
        j0 = k * 2
        _wait_in(rows0, dch0, semr0, semd0)
        pltpu.sync_copy(rows0, acc.at[dch0], add=True)
        _start(j0 + 2, rows0, dch0, semr0, semd0)
        _wait_in(rows1, dch1, semr1, semd1)
        pltpu.sync_copy(rows1, acc.at[dch1], add=True)
        _start(j0 + 3, rows1, dch1, semr1, semd1)
        return carry

    lax.fori_loop(0, N_CHUNKS // 2 - 1, _pair, 0)
    roff = _mo(N_CHUNKS * CHUNK)
    pltpu.async_copy(x_hbm.at[sidx.at[pl.ds(roff, REM)]], rowsr, semrr)
    pltpu.async_copy(edge_hbm.at[pl.ds(dbase + roff, REM)], dchr, semdr)
    _wait_in(rows0, dch0, semr0, semd0)
    pltpu.sync_copy(rows0, acc.at[dch0], add=True)
    _wait_in(rows1, dch1, semr1, semd1)
    pltpu.sync_copy(rows1, acc.at[dch1], add=True)
    pltpu.make_async_copy(x_hbm.at[sidx.at[pl.ds(0, REM)]], rowsr, semrr).wait()
    pltpu.make_async_copy(edge_hbm.at[pl.ds(0, REM)], dchr, semdr).wait()
    pltpu.sync_copy(rowsr, acc.at[dchr], add=True)
    plsc.subcore_barrier()

    pltpu.sync_copy(acc.at[pl.ds(abase, ROWS_PER_TILE), :],
                    out_hbm.at[c, pl.ds(abase, ROWS_PER_TILE), :])


_sc_segsum = pl.kernel(
    _sc_body,
    out_type=jax.ShapeDtypeStruct((N_CORES, N_PAD, D), jnp.float32),
    mesh=plsc.VectorSubcoreMesh(core_axis_name="c", subcore_axis_name="s"),
    scratch_types=[
        pltpu.VMEM((EDGES_PER_TILE,), jnp.int32),
        pltpu.VMEM((CHUNK,), jnp.int32),
        pltpu.VMEM((CHUNK,), jnp.int32),
        pltpu.VMEM((REM,), jnp.int32),
        pltpu.VMEM((CHUNK, D), jnp.float32),
        pltpu.VMEM((CHUNK, D), jnp.float32),
        pltpu.VMEM((REM, D), jnp.float32),
        pltpu.VMEM_SHARED((N_PAD, D), jnp.float32),
        pltpu.SemaphoreType.DMA,
        pltpu.SemaphoreType.DMA,
        pltpu.SemaphoreType.DMA,
        pltpu.SemaphoreType.DMA,
        pltpu.SemaphoreType.DMA,
        pltpu.SemaphoreType.DMA,
        pltpu.SemaphoreType.DMA,
        pltpu.SemaphoreType.DMA,
    ],
)


def _mm_body(p_ref, w_ref, b_ref, o_ref):
    h = p_ref[0] + p_ref[1]
    o_ref[...] = lax.dot_general(
        h, w_ref[...], (((1,), (1,)), ((), ())),
        preferred_element_type=jnp.float32) + b_ref[...]


ROW_BLK = 2000

_mm = pl.pallas_call(
    _mm_body,
    grid=(N_NODES // ROW_BLK,),
    in_specs=[
        pl.BlockSpec((N_CORES, ROW_BLK, D), lambda i: (0, i, 0)),
        pl.BlockSpec((D, D), lambda i: (0, 0)),
        pl.BlockSpec((1, D), lambda i: (0, 0)),
    ],
    out_specs=pl.BlockSpec((ROW_BLK, D), lambda i: (i, 0)),
    out_shape=jax.ShapeDtypeStruct((N_NODES, D), jnp.float32),
)


@jax.jit
def kernel(x, edge_index, W, b):
    edge_flat = edge_index.reshape(2 * N_EDGES)
    parts = _sc_segsum(x, edge_flat)
    return _mm(parts, W, b.reshape(1, D))
